# baseline (speedup 1.0000x reference)
import jax
import jax.numpy as jnp
from jax import lax
from jax.experimental import pallas as pl
from jax.experimental.pallas import tpu as pltpu
from jax.experimental.pallas import tpu_sc as plsc

B, L, D = 4096, 200, 64
N_ROWS = B * L

_info = plsc.get_sparse_core_info()
NC, NS = _info.num_cores, _info.num_subcores
NW = NC * NS
ROWS_PER_W = N_ROWS // NW
SEQ_PER_CHUNK = 4
CHUNK = SEQ_PER_CHUNK * L
N_CHUNKS = ROWS_PER_W // CHUNK
LANES = 16


def _body(idx_hbm, tok_hbm, pos_hbm, out_hbm, pos_v, idx_v, rows_v, sem):
    wid = lax.axis_index("s") * NC + lax.axis_index("c")
    pltpu.sync_copy(pos_hbm, pos_v)

    def chunk_body(c, carry):
        base = wid * ROWS_PER_W + c * CHUNK
        pltpu.sync_copy(idx_hbm.at[pl.ds(base, CHUNK)], idx_v)
        pltpu.async_copy(tok_hbm.at[idx_v], rows_v, sem).wait()

        def add_body(r, carry2):
            for s in range(SEQ_PER_CHUNK):
                for k in range(D // LANES):
                    sl = pl.ds(k * LANES, LANES)
                    rows_v[s * L + r, sl] = rows_v[s * L + r, sl] + pos_v[r, sl]
            return carry2

        lax.fori_loop(0, L, add_body, None)
        pltpu.sync_copy(rows_v, out_hbm.at[pl.ds(base, CHUNK)])
        return carry

    lax.fori_loop(0, N_CHUNKS, chunk_body, None)


def kernel(inputs, token_table, pos_table):
    idx = inputs.reshape(-1).astype(jnp.int32)
    out = pl.kernel(
        _body,
        out_type=jax.ShapeDtypeStruct((N_ROWS, D), jnp.float32),
        mesh=plsc.VectorSubcoreMesh(core_axis_name="c", subcore_axis_name="s"),
        compiler_params=pltpu.CompilerParams(use_tc_tiling_on_sc=False),
        scratch_types=[
            pltpu.VMEM((L, D), jnp.float32),
            pltpu.VMEM((CHUNK,), jnp.int32),
            pltpu.VMEM((CHUNK, D), jnp.float32),
            pltpu.SemaphoreType.DMA,
        ],
    )(idx, token_table, pos_table)
    return out.reshape(B, L, D)


# speedup vs baseline: 1.0940x; 1.0940x over previous
import jax
import jax.numpy as jnp
from jax import lax
from jax.experimental import pallas as pl
from jax.experimental.pallas import tpu as pltpu
from jax.experimental.pallas import tpu_sc as plsc

B, L, D = 4096, 200, 64
N_ROWS = B * L

_info = plsc.get_sparse_core_info()
NC, NS = _info.num_cores, _info.num_subcores
NW = NC * NS
ROWS_PER_W = N_ROWS // NW
SEQ_PER_CHUNK = 2
CHUNK = SEQ_PER_CHUNK * L
N_CHUNKS = ROWS_PER_W // CHUNK
LANES = 16


def _body(idx_hbm, tok_hbm, pos_hbm, out_hbm, pos_v, idx_v, rows0, rows1,
          gsem0, gsem1, osem):
    wid = lax.axis_index("s") * NC + lax.axis_index("c")
    base_w = wid * ROWS_PER_W
    pltpu.sync_copy(pos_hbm, pos_v)
    pltpu.sync_copy(idx_hbm.at[pl.ds(base_w, ROWS_PER_W)], idx_v)

    rows = (rows0, rows1)
    gsems = (gsem0, gsem1)

    def start_gather(c, b):
        pltpu.async_copy(
            tok_hbm.at[idx_v.at[pl.ds(c * CHUNK, CHUNK)]], rows[b], gsems[b])

    def wait_gather(c, b):
        pltpu.make_async_copy(
            tok_hbm.at[idx_v.at[pl.ds(c * CHUNK, CHUNK)]], rows[b],
            gsems[b]).wait()

    def start_out(c, b):
        pltpu.async_copy(
            rows[b], out_hbm.at[pl.ds(base_w + c * CHUNK, CHUNK)], osem)

    def wait_out(c, b):
        pltpu.make_async_copy(
            rows[b], out_hbm.at[pl.ds(base_w + c * CHUNK, CHUNK)],
            osem).wait()

    def add_pos(b):
        buf = rows[b]

        def add_body(r, carry):
            for s in range(SEQ_PER_CHUNK):
                for k in range(D // LANES):
                    sl = pl.ds(k * LANES, LANES)
                    buf[s * L + r, sl] = buf[s * L + r, sl] + pos_v[r, sl]
            return carry

        lax.fori_loop(0, L, add_body, None)

    start_gather(0, 0)
    start_gather(1, 1)
    wait_gather(0, 0)
    add_pos(0)
    start_out(0, 0)

    def group(g, carry):
        c1 = 2 * g + 1
        wait_out(c1 - 1, 0)
        start_gather(c1 + 1, 0)
        wait_gather(c1, 1)
        add_pos(1)
        start_out(c1, 1)

        c2 = 2 * g + 2
        wait_out(c2 - 1, 1)
        start_gather(c2 + 1, 1)
        wait_gather(c2, 0)
        add_pos(0)
        start_out(c2, 0)
        return carry

    lax.fori_loop(0, N_CHUNKS // 2 - 1, group, None)

    last = N_CHUNKS - 1
    wait_gather(last, 1)
    add_pos(1)
    start_out(last, 1)
    wait_out(last - 1, 0)
    wait_out(last, 1)


def kernel(inputs, token_table, pos_table):
    idx = inputs.reshape(-1).astype(jnp.int32)
    out = pl.kernel(
        _body,
        out_type=jax.ShapeDtypeStruct((N_ROWS, D), jnp.float32),
        mesh=plsc.VectorSubcoreMesh(core_axis_name="c", subcore_axis_name="s"),
        compiler_params=pltpu.CompilerParams(use_tc_tiling_on_sc=False),
        scratch_types=[
            pltpu.VMEM((L, D), jnp.float32),
            pltpu.VMEM((ROWS_PER_W,), jnp.int32),
            pltpu.VMEM((CHUNK, D), jnp.float32),
            pltpu.VMEM((CHUNK, D), jnp.float32),
            pltpu.SemaphoreType.DMA,
            pltpu.SemaphoreType.DMA,
            pltpu.SemaphoreType.DMA,
        ],
    )(idx, token_table, pos_table)
    return out.reshape(B, L, D)
